# baseline (speedup 1.0000x reference)
import jax
import jax.numpy as jnp
from jax import lax
from jax.experimental import pallas as pl
from jax.experimental.pallas import tpu as pltpu
from jax.experimental.pallas import tpu_sc as plsc

_NC = 2
_NS = 16
_NW = _NC * _NS

_REP = 128


def _tc_add_body(geom_ref, tok_ref, out_ref):
    out_ref[...] = tok_ref[...] + geom_ref[...][:, None, :]


def _make_sc_geom(b, d, bpw):
    mesh = plsc.VectorSubcoreMesh(
        core_axis_name="c", subcore_axis_name="s",
        num_cores=_NC, num_subcores=_NS)

    def sc_geom(vids, sids, ctable_rep):
        @pl.kernel(
            out_type=jax.ShapeDtypeStruct((b, d), jnp.float32),
            mesh=mesh,
            scratch_types=[
                pltpu.VMEM((bpw,), jnp.int32),
                pltpu.VMEM((bpw,), jnp.int32),
                pltpu.VMEM((bpw, d), jnp.float32),
                pltpu.SemaphoreType.DMA,
            ],
        )
        def run(vids_hbm, sids_hbm, ctable_hbm, geom_hbm,
                v_v, s_v, rows_v, sem):
            wid = lax.axis_index("s") * _NC + lax.axis_index("c")
            base = wid * bpw
            h1 = pltpu.async_copy(vids_hbm.at[pl.ds(base, bpw)], v_v, sem)
            h2 = pltpu.async_copy(sids_hbm.at[pl.ds(base, bpw)], s_v, sem)
            h1.wait()
            h2.wait()
            lane = lax.iota(jnp.int32, 16)
            for i in range(bpw // 16):
                s = pl.ds(i * 16, 16)
                rep = (base + i * 16 + lane) & (_REP - 1)
                v_v[s] = v_v[s] * 2 + s_v[s] + rep * 4
            pltpu.async_copy(ctable_hbm.at[v_v], rows_v, sem).wait()
            pltpu.sync_copy(rows_v, geom_hbm.at[pl.ds(base, bpw)])

        return run(vids, sids, ctable_rep)

    return sc_geom


def kernel(tokens, view_ids, side_ids, view_embed, side_embed):
    B, L, D = tokens.shape
    BB = 128
    NB = B // BB
    BPW = B // _NW

    ctable_rep = (view_embed[None, :, None, :]
                  + side_embed[None, None, :, :])
    ctable_rep = jnp.broadcast_to(ctable_rep, (_REP, 2, 2, D)).reshape(-1, D)

    geom = _make_sc_geom(B, D, BPW)(view_ids.astype(jnp.int32),
                                    side_ids.astype(jnp.int32), ctable_rep)

    return pl.pallas_call(
        _tc_add_body,
        grid=(NB,),
        in_specs=[
            pl.BlockSpec((BB, D), lambda i: (i, 0)),
            pl.BlockSpec((BB, L, D), lambda i: (i, 0, 0)),
        ],
        out_specs=pl.BlockSpec((BB, L, D), lambda i: (i, 0, 0)),
        out_shape=jax.ShapeDtypeStruct((B, L, D), tokens.dtype),
    )(geom, tokens)


# speedup vs baseline: 1.0055x; 1.0055x over previous
import jax
import jax.numpy as jnp
from jax import lax
from jax.experimental import pallas as pl
from jax.experimental.pallas import tpu as pltpu
from jax.experimental.pallas import tpu_sc as plsc

_NC = 2
_NS = 16
_NW = _NC * _NS

_REP = 512


def _tc_add_body(geom_ref, tok_ref, out_ref):
    out_ref[...] = tok_ref[...] + geom_ref[...][:, None, :]


def _make_sc_geom(b, d, bpw):
    mesh = plsc.VectorSubcoreMesh(
        core_axis_name="c", subcore_axis_name="s",
        num_cores=_NC, num_subcores=_NS)

    def sc_geom(vids, sids, ctable_rep):
        @pl.kernel(
            out_type=jax.ShapeDtypeStruct((b, d), jnp.float32),
            mesh=mesh,
            scratch_types=[
                pltpu.VMEM((bpw,), jnp.int32),
                pltpu.VMEM((bpw,), jnp.int32),
                pltpu.VMEM((bpw, d), jnp.float32),
                pltpu.SemaphoreType.DMA,
            ],
        )
        def run(vids_hbm, sids_hbm, ctable_hbm, geom_hbm,
                v_v, s_v, rows_v, sem):
            wid = lax.axis_index("s") * _NC + lax.axis_index("c")
            base = wid * bpw
            h1 = pltpu.async_copy(vids_hbm.at[pl.ds(base, bpw)], v_v, sem)
            h2 = pltpu.async_copy(sids_hbm.at[pl.ds(base, bpw)], s_v, sem)
            h1.wait()
            h2.wait()
            lane = lax.iota(jnp.int32, 16)
            for i in range(bpw // 16):
                s = pl.ds(i * 16, 16)
                rep = (base + i * 16 + lane) & (_REP - 1)
                v_v[s] = v_v[s] * 2 + s_v[s] + rep * 4
            pltpu.async_copy(ctable_hbm.at[v_v], rows_v, sem).wait()
            pltpu.sync_copy(rows_v, geom_hbm.at[pl.ds(base, bpw)])

        return run(vids, sids, ctable_rep)

    return sc_geom


def kernel(tokens, view_ids, side_ids, view_embed, side_embed):
    B, L, D = tokens.shape
    BB = 128
    NB = B // BB
    BPW = B // _NW

    ctable_rep = (view_embed[None, :, None, :]
                  + side_embed[None, None, :, :])
    ctable_rep = jnp.broadcast_to(ctable_rep, (_REP, 2, 2, D)).reshape(-1, D)

    geom = _make_sc_geom(B, D, BPW)(view_ids.astype(jnp.int32),
                                    side_ids.astype(jnp.int32), ctable_rep)

    return pl.pallas_call(
        _tc_add_body,
        grid=(NB,),
        in_specs=[
            pl.BlockSpec((BB, D), lambda i: (i, 0)),
            pl.BlockSpec((BB, L, D), lambda i: (i, 0, 0)),
        ],
        out_specs=pl.BlockSpec((BB, L, D), lambda i: (i, 0, 0)),
        out_shape=jax.ShapeDtypeStruct((B, L, D), tokens.dtype),
    )(geom, tokens)
